# baseline (speedup 1.0000x reference)
import functools

import jax
import jax.numpy as jnp
from jax import lax
from jax.experimental import pallas as pl
from jax.experimental.pallas import tpu as pltpu
from jax.experimental.pallas import tpu_sc as plsc

E = 16
TOPK = 2
D = 1024
U = 1024
N = 4096
S = N * TOPK
B = 256
NT = S // B + E
SP = NT * B

_SC_CORES = 2
_SC_SUBCORES = 16
_SC_WORKERS = _SC_CORES * _SC_SUBCORES
_CHUNK = 64


def _router_body(x_ref, wr_ref, br_ref, w_ref, i_ref):
    x = x_ref[...]
    logits = jnp.dot(x, wr_ref[...], preferred_element_type=jnp.float32)
    logits = logits + br_ref[...]
    probs = jax.nn.softmax(logits, axis=-1)
    col = lax.broadcasted_iota(jnp.int32, probs.shape, 1)
    m1 = jnp.max(probs, axis=1, keepdims=True)
    i1 = jnp.min(jnp.where(probs == m1, col, E), axis=1, keepdims=True)
    p2 = jnp.where(col == i1, -1.0, probs)
    m2 = jnp.max(p2, axis=1, keepdims=True)
    i2 = jnp.min(jnp.where(p2 == m2, col, E), axis=1, keepdims=True)
    w_ref[...] = jnp.concatenate([m1, m2], axis=1)
    i_ref[...] = jnp.concatenate([i1, i2], axis=1)


def _router(x, Wr, br):
    TB = 512
    return pl.pallas_call(
        _router_body,
        grid=(N // TB,),
        in_specs=[
            pl.BlockSpec((TB, D), lambda t: (t, 0)),
            pl.BlockSpec((D, E), lambda t: (0, 0)),
            pl.BlockSpec((1, E), lambda t: (0, 0)),
        ],
        out_specs=[
            pl.BlockSpec((TB, TOPK), lambda t: (t, 0)),
            pl.BlockSpec((TB, TOPK), lambda t: (t, 0)),
        ],
        out_shape=[
            jax.ShapeDtypeStruct((N, TOPK), jnp.float32),
            jax.ShapeDtypeStruct((N, TOPK), jnp.int32),
        ],
    )(x, Wr, br.reshape(1, E))


def _gmm_body(meta_ref, x_ref, we_ref, be_ref, h_ref):
    t = pl.program_id(0)

    @pl.when(t < meta_ref[NT])
    def _():
        acc = jnp.dot(x_ref[...], we_ref[0], preferred_element_type=jnp.float32)
        h_ref[...] = jnp.maximum(acc + be_ref[0].reshape(1, U), 0.0)


def _gmm(meta, x_padded, We, be):
    grid_spec = pltpu.PrefetchScalarGridSpec(
        num_scalar_prefetch=1,
        grid=(NT,),
        in_specs=[
            pl.BlockSpec((B, D), lambda t, m: (t, 0)),
            pl.BlockSpec((1, D, U), lambda t, m: (m[t], 0, 0)),
            pl.BlockSpec((1, 8, U // 8), lambda t, m: (m[t], 0, 0)),
        ],
        out_specs=pl.BlockSpec((B, U), lambda t, m: (t, 0)),
    )
    return pl.pallas_call(
        _gmm_body,
        grid_spec=grid_spec,
        out_shape=jax.ShapeDtypeStruct((SP, U), jnp.float32),
    )(meta, x_padded, We, be.reshape(E, 8, U // 8))


def _sc_gather(table, idx):
    n_idx = idx.shape[0]
    d = table.shape[1]
    rows_per_w = n_idx // _SC_WORKERS
    n_chunks = rows_per_w // _CHUNK
    mesh = plsc.VectorSubcoreMesh(core_axis_name="c", subcore_axis_name="s")

    @functools.partial(
        pl.kernel,
        mesh=mesh,
        out_type=jax.ShapeDtypeStruct((n_idx, d), jnp.float32),
        scratch_types=[
            pltpu.VMEM((_CHUNK,), jnp.int32),
            pltpu.VMEM((_CHUNK, d), jnp.float32),
            pltpu.SemaphoreType.DMA,
        ],
    )
    def k(table_hbm, idx_hbm, out_hbm, idx_v, rows_v, sem):
        wid = lax.axis_index("s") * _SC_CORES + lax.axis_index("c")
        base = wid * rows_per_w
        for c in range(n_chunks):
            o = base + c * _CHUNK
            pltpu.sync_copy(idx_hbm.at[pl.ds(o, _CHUNK)], idx_v)
            pltpu.async_copy(table_hbm.at[idx_v], rows_v, sem).wait()
            pltpu.sync_copy(rows_v, out_hbm.at[pl.ds(o, _CHUNK)])

    return k(table, idx)


def _combine_body(h2_ref, w_ref, o_ref):
    a = h2_ref[...]
    w = w_ref[...]
    o_ref[...] = a[:, :U] * w[:, 0:1] + a[:, U:] * w[:, 1:2]


def _combine(h_pair, w2):
    TB = 256
    return pl.pallas_call(
        _combine_body,
        grid=(N // TB,),
        in_specs=[
            pl.BlockSpec((TB, 2 * U), lambda t: (t, 0)),
            pl.BlockSpec((TB, TOPK), lambda t: (t, 0)),
        ],
        out_specs=pl.BlockSpec((TB, U), lambda t: (t, 0)),
        out_shape=jax.ShapeDtypeStruct((N, U), jnp.float32),
    )(h_pair.reshape(N, 2 * U), w2)


def kernel(inputs, Wr, br, We, be):
    x = inputs
    w2, i2 = _router(x, Wr, br)

    flat_e = i2.reshape(-1)
    sort_idx = jnp.argsort(flat_e).astype(jnp.int32)
    e_sorted = jnp.take(flat_e, sort_idx)
    tok_sorted = (sort_idx // TOPK).astype(jnp.int32)
    counts = jnp.bincount(flat_e, length=E).astype(jnp.int32)
    off = jnp.concatenate([jnp.zeros((1,), jnp.int32),
                           jnp.cumsum(counts)[:-1].astype(jnp.int32)])
    tiles_e = (counts + B - 1) // B
    tile_end = jnp.cumsum(tiles_e).astype(jnp.int32)
    tile_off = tile_end - tiles_e
    total_tiles = tile_end[-1]
    t_ar = jnp.arange(NT, dtype=jnp.int32)
    te_raw = jnp.clip(
        jnp.searchsorted(tile_end, t_ar, side="right"), 0, E - 1
    ).astype(jnp.int32)
    e_last = e_sorted[-1]
    te = jnp.where(t_ar < total_tiles, te_raw, e_last)
    meta = jnp.concatenate([te, total_tiles[None]])

    q = jnp.arange(SP, dtype=jnp.int32)
    qt = q // B
    eq = te[qt]
    local = (qt - tile_off[eq]) * B + (q % B)
    validq = (local < counts[eq]) & (qt < total_tiles)
    src_tok = jnp.where(
        validq, jnp.take(tok_sorted, jnp.clip(off[eq] + local, 0, S - 1)), 0
    ).astype(jnp.int32)

    p = jnp.arange(S, dtype=jnp.int32)
    dst_padded = (tile_off[e_sorted] * B + (p - off[e_sorted])).astype(jnp.int32)
    comb_src = jnp.zeros((S,), jnp.int32).at[sort_idx].set(dst_padded)

    x_padded = _sc_gather(x, src_tok)
    h_padded = _gmm(meta, x_padded, We, be)
    h_pair = _sc_gather(h_padded, comb_src)
    return _combine(h_pair, w2)
